# baseline (speedup 1.0000x reference)
import functools

import jax
import jax.numpy as jnp
from jax import lax
from jax.experimental import pallas as pl
from jax.experimental.pallas import tpu as pltpu
from jax.experimental.pallas import tpu_sc as plsc

B, NF, NT, V, D, L = 4096, 10, 26, 100000, 32, 20
DENSE_OUT, OVER_OUT = 32, 16

_NC = 2
_NS = 16
_NW = _NC * _NS

_PAIRS = B * NT
_PPW = _PAIRS // _NW
_CP = 64
_NCH = _PPW // _CP
_SL = 128
_RS = _CP * L // _SL
_GCH = _PAIRS // _CP

@functools.cache
def _get_sc_pool():
    mesh = plsc.VectorSubcoreMesh(core_axis_name="c", subcore_axis_name="s")
    return functools.partial(
        pl.kernel,
        mesh=mesh,
        compiler_params=pltpu.CompilerParams(use_tc_tiling_on_sc=False),
        out_type=jax.ShapeDtypeStruct((_PAIRS, D), jnp.float32),
        scratch_types=[
            pltpu.VMEM((2, _RS, _SL), jnp.int32),
            pltpu.VMEM((2, _RS, _SL, D), jnp.float32),
            pltpu.VMEM((2, _CP, D), jnp.float32),
            pltpu.SemaphoreType.DMA,
            pltpu.SemaphoreType.DMA,
        ],
    )(_sc_pool_body)


def _sc_pool_body(tables_hbm, idx_hbm, out_hbm, idx_v, rows_v, out_v, sem0, sem1):
    wid = lax.axis_index("s") * _NC + lax.axis_index("c")
    sems = (sem0, sem1)

    def start(slot, ci, sem):
        pltpu.sync_copy(idx_hbm.at[ci], idx_v.at[slot])
        for j in range(_RS):
            pltpu.async_copy(tables_hbm.at[idx_v.at[slot, j]],
                             rows_v.at[slot, j], sem)

    def drain(slot, sem):
        for j in range(_RS):
            pltpu.make_async_copy(tables_hbm.at[idx_v.at[slot, j]],
                                  rows_v.at[slot, j], sem).wait()

    def compute(slot):
        def pair_body(p, carry):
            base = p * L
            acc_a = rows_v[slot, base >> 7, base & 127, pl.ds(0, 16)]
            acc_b = rows_v[slot, base >> 7, base & 127, pl.ds(16, 16)]
            for l in range(1, L):
                r = base + l
                j = r >> 7
                k = r & 127
                acc_a = acc_a + rows_v[slot, j, k, pl.ds(0, 16)]
                acc_b = acc_b + rows_v[slot, j, k, pl.ds(16, 16)]
            out_v[slot, p, pl.ds(0, 16)] = acc_a
            out_v[slot, p, pl.ds(16, 16)] = acc_b
            return carry

        lax.fori_loop(0, _CP, pair_body, 0)

    chunk0 = wid * _NCH
    start(0, chunk0, sems[0])
    start(1, chunk0 + 1, sems[1])

    def loop_body(c2, carry):
        for slot in range(2):
            ci = c2 * 2 + slot
            drain(slot, sems[slot])
            compute(slot)
            pltpu.sync_copy(
                out_v.at[slot],
                out_hbm.at[pl.ds((chunk0 + ci) * _CP, _CP)])
            nxt = ci + 2

            @pl.when(nxt < _NCH)
            def _():
                start(slot, chunk0 + nxt, sems[slot])
        return carry

    lax.fori_loop(0, _NCH // 2, loop_body, 0)


_TV = V


_QS = 24960
_Q = 25120
_LC = 3584
_QF = 7 * _LC
_LIN_UNITS = [(a, i * _LC) for a in range(4) for i in range(7)]


def _lin_body(tin, tl, o, scr, sem0, sem1):
    t = pl.program_id(0)
    sems = (sem0, sem1)

    def copy(u, slot):
        a, r0 = _LIN_UNITS[u]
        return pltpu.make_async_copy(
            tin.at[t, :, pl.ds(a * _QS + r0, _LC)],
            scr.at[slot], sems[slot])

    copy(0, 0).start()
    copy(1, 1).start()
    for u, (a, r0) in enumerate(_LIN_UNITS):
        slot = u & 1
        copy(u, slot).wait()
        o[pl.ds(r0, _LC), a * D:(a + 1) * D] = jnp.transpose(scr[slot])
        if u + 2 < len(_LIN_UNITS):
            copy(u + 2, slot).start()

    yt = jnp.transpose(tl[0])
    for a in range(4):
        o[pl.ds(_QF, 32), a * D:(a + 1) * D] = yt[a * D:(a + 1) * D, :]


_linearize = pl.pallas_call(
    _lin_body,
    grid=(NT,),
    in_specs=[pl.BlockSpec(memory_space=pl.ANY),
              pl.BlockSpec((1, D, 128), lambda t: (t, 0, 0))],
    out_specs=pl.BlockSpec((_Q, 128), lambda t: (t, 0)),
    out_shape=jax.ShapeDtypeStruct((NT * _Q, 128), jnp.float32),
    scratch_shapes=[
        pltpu.VMEM((2, D, _LC), jnp.float32),
        pltpu.SemaphoreType.DMA,
        pltpu.SemaphoreType.DMA,
    ],
)


_BM = 512


def _head_body(ff, dw, db, pooled, owd, ows, ob, o):
    dense = jnp.maximum(
        jnp.dot(ff[:], dw[:], preferred_element_type=jnp.float32) + db[:], 0.0)
    o[:] = (jnp.dot(dense, owd[:], preferred_element_type=jnp.float32)
            + jnp.dot(pooled[:], ows[:], preferred_element_type=jnp.float32)
            + ob[:])


_tc_head = pl.pallas_call(
    _head_body,
    grid=(B // _BM,),
    in_specs=[
        pl.BlockSpec((_BM, NF), lambda i: (i, 0)),
        pl.BlockSpec((NF, DENSE_OUT), lambda i: (0, 0)),
        pl.BlockSpec((1, DENSE_OUT), lambda i: (0, 0)),
        pl.BlockSpec((_BM, NT * D), lambda i: (i, 0)),
        pl.BlockSpec((DENSE_OUT, OVER_OUT), lambda i: (0, 0)),
        pl.BlockSpec((NT * D, OVER_OUT), lambda i: (0, 0)),
        pl.BlockSpec((1, OVER_OUT), lambda i: (0, 0)),
    ],
    out_specs=pl.BlockSpec((_BM, OVER_OUT), lambda i: (i, 0)),
    out_shape=jax.ShapeDtypeStruct((B, OVER_OUT), jnp.float32),
)


def kernel(float_features, indices, tables, dense_w, dense_b, over_w, over_b):
    tables_t = jnp.transpose(tables, (0, 2, 1))
    tails = jnp.concatenate(
        [tables_t[:, :, a * _QS + _QF:a * _QS + _Q] for a in range(4)], axis=2)
    tables2d = _linearize(tables_t, tails).reshape(NT * _Q * 4, D)
    idx32 = indices.astype(jnp.int32)
    a = ((idx32 >= _QS).astype(jnp.int32)
         + (idx32 >= 2 * _QS).astype(jnp.int32)
         + (idx32 >= 3 * _QS).astype(jnp.int32))
    r = idx32 - a * _QS
    toff = (jnp.arange(NT, dtype=jnp.int32) * _Q)[None, :, None]
    flat_idx = ((toff + r) << 2) + a
    idx_chunks = flat_idx.reshape(_GCH, _RS, _SL)
    pooled = _get_sc_pool()(tables2d, idx_chunks)
    pooled2 = pooled.reshape(B, NT * D)
    out = _tc_head(float_features, dense_w, dense_b.reshape(1, DENSE_OUT),
                   pooled2, over_w[:DENSE_OUT], over_w[DENSE_OUT:],
                   over_b.reshape(1, OVER_OUT))
    return out


# speedup vs baseline: 2.2094x; 2.2094x over previous
import functools

import jax
import jax.numpy as jnp
from jax import lax
from jax.experimental import pallas as pl
from jax.experimental.pallas import tpu as pltpu
from jax.experimental.pallas import tpu_sc as plsc

B, NF, NT, V, D, L = 4096, 10, 26, 100000, 32, 20
DENSE_OUT, OVER_OUT = 32, 16

_NC = 2
_NS = 16
_NW = _NC * _NS

_PAIRS = B * NT
_PPW = _PAIRS // _NW
_CP = 64
_NCH = _PPW // _CP
_SL = 128
_RS = _CP * L // _SL
_GCH = _PAIRS // _CP

@functools.cache
def _get_sc_pool():
    mesh = plsc.VectorSubcoreMesh(core_axis_name="c", subcore_axis_name="s")
    return functools.partial(
        pl.kernel,
        mesh=mesh,
        compiler_params=pltpu.CompilerParams(use_tc_tiling_on_sc=False),
        out_type=jax.ShapeDtypeStruct((_PAIRS, D), jnp.float32),
        scratch_types=[
            pltpu.VMEM((2, _RS, _SL), jnp.int32),
            pltpu.VMEM((2, _RS, _SL, D), jnp.float32),
            pltpu.VMEM((2, _CP, D), jnp.float32),
            pltpu.SemaphoreType.DMA,
            pltpu.SemaphoreType.DMA,
        ],
    )(_sc_pool_body)


def _sc_pool_body(tables_hbm, idx_hbm, out_hbm, idx_v, rows_v, out_v, sem0, sem1):
    wid = lax.axis_index("s") * _NC + lax.axis_index("c")
    sems = (sem0, sem1)

    def start(slot, ci, sem):
        pltpu.sync_copy(idx_hbm.at[ci], idx_v.at[slot])
        for j in range(_RS):
            pltpu.async_copy(tables_hbm.at[idx_v.at[slot, j]],
                             rows_v.at[slot, j], sem)

    def drain(slot, sem):
        for j in range(_RS):
            pltpu.make_async_copy(tables_hbm.at[idx_v.at[slot, j]],
                                  rows_v.at[slot, j], sem).wait()

    def compute(slot):
        def pair_body(p, carry):
            base = p * L
            acc_a = rows_v[slot, base >> 7, base & 127, pl.ds(0, 16)]
            acc_b = rows_v[slot, base >> 7, base & 127, pl.ds(16, 16)]
            for l in range(1, L):
                r = base + l
                j = r >> 7
                k = r & 127
                acc_a = acc_a + rows_v[slot, j, k, pl.ds(0, 16)]
                acc_b = acc_b + rows_v[slot, j, k, pl.ds(16, 16)]
            out_v[slot, p, pl.ds(0, 16)] = acc_a
            out_v[slot, p, pl.ds(16, 16)] = acc_b
            return carry

        lax.fori_loop(0, _CP, pair_body, 0)

    chunk0 = wid * _NCH
    start(0, chunk0, sems[0])
    start(1, chunk0 + 1, sems[1])

    def loop_body(c2, carry):
        for slot in range(2):
            ci = c2 * 2 + slot
            drain(slot, sems[slot])
            compute(slot)
            pltpu.sync_copy(
                out_v.at[slot],
                out_hbm.at[pl.ds((chunk0 + ci) * _CP, _CP)])
            nxt = ci + 2

            @pl.when(nxt < _NCH)
            def _():
                start(slot, chunk0 + nxt, sems[slot])
        return carry

    lax.fori_loop(0, _NCH // 2, loop_body, 0)


_TV = V


_QS = 24960
_Q = 25120
_LC = 3584
_QF = 7 * _LC
_LIN_UNITS = [(a, i * _LC) for a in range(4) for i in range(7)]


def _lin_body(tin, tl, o):
    x = tin[0]
    for i in range(7):
        r0 = i * _LC
        m = jnp.concatenate(
            [x[:, a * _QS + r0:a * _QS + r0 + _LC] for a in range(4)],
            axis=0)
        o[pl.ds(r0, _LC), :] = jnp.transpose(m)
    mt = jnp.concatenate(
        [tl[0][:, a * D:(a + 1) * D] for a in range(4)], axis=0)
    o[pl.ds(_QF, 32), :] = jnp.transpose(mt)


_linearize = pl.pallas_call(
    _lin_body,
    grid=(NT,),
    in_specs=[pl.BlockSpec((1, D, _TV), lambda t: (t, 0, 0)),
              pl.BlockSpec((1, D, 128), lambda t: (t, 0, 0))],
    out_specs=pl.BlockSpec((_Q, 128), lambda t: (t, 0)),
    out_shape=jax.ShapeDtypeStruct((NT * _Q, 128), jnp.float32),
    compiler_params=pltpu.CompilerParams(vmem_limit_bytes=65011712),
)


_BM = 512


def _head_body(ff, dw, db, pooled, owd, ows, ob, o):
    dense = jnp.maximum(
        jnp.dot(ff[:], dw[:], preferred_element_type=jnp.float32) + db[:], 0.0)
    o[:] = (jnp.dot(dense, owd[:], preferred_element_type=jnp.float32)
            + jnp.dot(pooled[:], ows[:], preferred_element_type=jnp.float32)
            + ob[:])


_tc_head = pl.pallas_call(
    _head_body,
    grid=(B // _BM,),
    in_specs=[
        pl.BlockSpec((_BM, NF), lambda i: (i, 0)),
        pl.BlockSpec((NF, DENSE_OUT), lambda i: (0, 0)),
        pl.BlockSpec((1, DENSE_OUT), lambda i: (0, 0)),
        pl.BlockSpec((_BM, NT * D), lambda i: (i, 0)),
        pl.BlockSpec((DENSE_OUT, OVER_OUT), lambda i: (0, 0)),
        pl.BlockSpec((NT * D, OVER_OUT), lambda i: (0, 0)),
        pl.BlockSpec((1, OVER_OUT), lambda i: (0, 0)),
    ],
    out_specs=pl.BlockSpec((_BM, OVER_OUT), lambda i: (i, 0)),
    out_shape=jax.ShapeDtypeStruct((B, OVER_OUT), jnp.float32),
)


def kernel(float_features, indices, tables, dense_w, dense_b, over_w, over_b):
    tables_t = jnp.transpose(tables, (0, 2, 1))
    tails = jnp.concatenate(
        [tables_t[:, :, a * _QS + _QF:a * _QS + _Q] for a in range(4)], axis=2)
    tables2d = _linearize(tables_t, tails).reshape(NT * _Q * 4, D)
    idx32 = indices.astype(jnp.int32)
    a = ((idx32 >= _QS).astype(jnp.int32)
         + (idx32 >= 2 * _QS).astype(jnp.int32)
         + (idx32 >= 3 * _QS).astype(jnp.int32))
    r = idx32 - a * _QS
    toff = (jnp.arange(NT, dtype=jnp.int32) * _Q)[None, :, None]
    flat_idx = ((toff + r) << 2) + a
    idx_chunks = flat_idx.reshape(_GCH, _RS, _SL)
    pooled = _get_sc_pool()(tables2d, idx_chunks)
    pooled2 = pooled.reshape(B, NT * D)
    out = _tc_head(float_features, dense_w, dense_b.reshape(1, DENSE_OUT),
                   pooled2, over_w[:DENSE_OUT], over_w[DENSE_OUT:],
                   over_b.reshape(1, OVER_OUT))
    return out
